# baseline (speedup 1.0000x reference)
import jax
import jax.numpy as jnp
from jax import lax
from jax.experimental import pallas as pl
from jax.experimental.pallas import tpu as pltpu
from jax.experimental.pallas import tpu_sc as plsc

N = 10000
E = 640000
D_IN = 128
D_H1 = 256
D_H2 = 128
D_OUT = 128
DH = 64

_NC = 2
_NS = 16
_CH = 128
_SS = 8
_NB = 4
_TSUB = 320
_NSUP = _TSUB // _SS
_EROWS = _NS * _TSUB
_NPAD = 10112
_DROWS = _NPAD // _NS
_SPAD = 10240
_SD = 640

_R = 632


def _offset_idx(idx_s, off):
    for jj in range(_SS):
        for k in range(_CH // 16):
            sl = pl.ds(k * 16, 16)
            idx_s[jj, sl] = idx_s[jj, sl] + off


def _spmm_sc_body(pcat_hbm, src_hbm, dst_hbm, zero_hbm, out_hbm,
                  src_s, dst_s, rows_v, acc_sh, sem0, sem1, sem2, sem3):
    c = lax.axis_index("c")
    s = lax.axis_index("s")
    sems = [sem0, sem1, sem2, sem3]
    pltpu.sync_copy(zero_hbm.at[pl.ds(s * _DROWS, _DROWS), :],
                    acc_sh.at[pl.ds(s * _DROWS, _DROWS), :])
    plsc.subcore_barrier()
    off = c * _NPAD

    def gather(j):
        b = j % _NB
        return pltpu.async_copy(pcat_hbm.at[src_s.at[j]], rows_v.at[b],
                                sems[b])

    def outer(g, carry):
        r0 = s * _TSUB + g * _SS
        pltpu.sync_copy(src_hbm.at[pl.ds(r0, _SS), :], src_s)
        pltpu.sync_copy(dst_hbm.at[pl.ds(r0, _SS), :], dst_s)
        _offset_idx(src_s, off)
        gd = [gather(j) for j in range(_NB)]
        for j in range(_SS):
            b = j % _NB
            gd[j].wait()
            pltpu.sync_copy(rows_v.at[b], acc_sh.at[dst_s.at[j]], add=True)
            if j + _NB < _SS:
                gd.append(gather(j + _NB))
        return carry

    lax.fori_loop(0, _NSUP, outer, 0)
    plsc.subcore_barrier()
    pltpu.sync_copy(acc_sh.at[pl.ds(s * _DROWS, _DROWS), :],
                    out_hbm.at[c, pl.ds(s * _DROWS, _DROWS), :])


@jax.jit
def _spmm_sc(pcat, src2d, dst2d, zeros2d):
    f = pl.kernel(
        _spmm_sc_body,
        out_type=jax.ShapeDtypeStruct((_NC, _NPAD, DH), jnp.float32),
        mesh=plsc.VectorSubcoreMesh(core_axis_name="c", subcore_axis_name="s"),
        compiler_params=pltpu.CompilerParams(needs_layout_passes=False, use_tc_tiling_on_sc=False),
        scratch_types=[
            pltpu.VMEM((_SS, _CH), jnp.int32),
            pltpu.VMEM((_SS, _CH), jnp.int32),
            pltpu.VMEM((_NB, _CH, DH), jnp.float32),
            pltpu.VMEM_SHARED((_NPAD, DH), jnp.float32),
            pltpu.SemaphoreType.DMA,
            pltpu.SemaphoreType.DMA,
            pltpu.SemaphoreType.DMA,
            pltpu.SemaphoreType.DMA,
        ],
    )
    return f(pcat, src2d, dst2d, zeros2d)


def _gat_sc_body(xcat_hbm, al_hbm, ar_hbm, src_hbm, dst_hbm, z2_hbm, z1_hbm,
                 agg_hbm, s0_hbm, d0_hbm,
                 al_v, ar_v, src_s, dst_s, raw_s, rows_v, exb, ones_v,
                 acc_sh, s_sh, deg_sh, sem0, sem1, sem2, sem3):
    c = lax.axis_index("c")
    s = lax.axis_index("s")
    sems = [sem0, sem1, sem2, sem3]
    pltpu.sync_copy(z2_hbm.at[pl.ds(s * _DROWS, _DROWS), :],
                    acc_sh.at[pl.ds(s * _DROWS, _DROWS), :])
    pltpu.sync_copy(z1_hbm.at[pl.ds(s * _SD, _SD)],
                    s_sh.at[pl.ds(s * _SD, _SD)])
    pltpu.sync_copy(z1_hbm.at[pl.ds(s * _SD, _SD)],
                    deg_sh.at[pl.ds(s * _SD, _SD)])
    pltpu.sync_copy(al_hbm, al_v)
    pltpu.sync_copy(ar_hbm, ar_v)
    for k in range(_CH // 16):
        ones_v[pl.ds(k * 16, 16)] = jnp.ones((16,), jnp.float32)
    plsc.subcore_barrier()
    off = c * N
    is_c0 = c == 0

    def gather(j):
        b = j % _NB
        return pltpu.async_copy(xcat_hbm.at[src_s.at[j]], rows_v.at[b],
                                sems[b])

    def outer(g, carry):
        r0 = s * _TSUB + g * _SS
        pltpu.sync_copy(src_hbm.at[pl.ds(r0, _SS), :], raw_s)
        pltpu.sync_copy(dst_hbm.at[pl.ds(r0, _SS), :], dst_s)
        for jj in range(_SS):
            for k in range(_CH // 16):
                sl = pl.ds(k * 16, 16)
                src_s[jj, sl] = raw_s[jj, sl] + off
        gd = [gather(j) for j in range(_NB)]
        for j in range(_SS):
            b = j % _NB
            gd[j].wait()
            for k in range(_CH // 16):
                sl = pl.ds(k * 16, 16)
                sv = raw_s[j, sl]
                dv = dst_s[j, sl]
                e = plsc.load_gather(al_v, [sv]) + plsc.load_gather(ar_v, [dv])
                e = jnp.maximum(e, 0.2 * e)
                exv = jnp.exp(e)
                exb[sl] = exv
                for l in range(16):
                    wl = exv[l]
                    r = k * 16 + l
                    for m in range(DH // 16):
                        slm = pl.ds(m * 16, 16)
                        rows_v[b, r, slm] = rows_v[b, r, slm] * wl

            @pl.when(is_c0)
            def _sdeg():
                pltpu.sync_copy(exb, s_sh.at[dst_s.at[j]], add=True)
                pltpu.sync_copy(ones_v, deg_sh.at[dst_s.at[j]], add=True)

            pltpu.sync_copy(rows_v.at[b], acc_sh.at[dst_s.at[j]], add=True)
            if j + _NB < _SS:
                gd.append(gather(j + _NB))
        return carry

    lax.fori_loop(0, _NSUP, outer, 0)
    plsc.subcore_barrier()
    pltpu.sync_copy(acc_sh.at[pl.ds(s * _DROWS, _DROWS), :],
                    agg_hbm.at[c, pl.ds(s * _DROWS, _DROWS), :])
    sl1 = pl.ds(s * _SD, _SD)

    @pl.when(is_c0)
    def _dump_sdeg():
        pltpu.sync_copy(s_sh.at[sl1], s0_hbm.at[sl1])
        pltpu.sync_copy(deg_sh.at[sl1], d0_hbm.at[sl1])


@jax.jit
def _gat_sc(xcat, al, ar, src2d, dst2d, zeros2d, zeros1d):
    f = pl.kernel(
        _gat_sc_body,
        out_type=[
            jax.ShapeDtypeStruct((_NC, _NPAD, DH), jnp.float32),
            jax.ShapeDtypeStruct((_SPAD,), jnp.float32),
            jax.ShapeDtypeStruct((_SPAD,), jnp.float32),
        ],
        mesh=plsc.VectorSubcoreMesh(core_axis_name="c", subcore_axis_name="s"),
        compiler_params=pltpu.CompilerParams(needs_layout_passes=False, use_tc_tiling_on_sc=False),
        scratch_types=[
            pltpu.VMEM((N,), jnp.float32),
            pltpu.VMEM((N,), jnp.float32),
            pltpu.VMEM((_SS, _CH), jnp.int32),
            pltpu.VMEM((_SS, _CH), jnp.int32),
            pltpu.VMEM((_SS, _CH), jnp.int32),
            pltpu.VMEM((_NB, _CH, DH), jnp.float32),
            pltpu.VMEM((_CH,), jnp.float32),
            pltpu.VMEM((_CH,), jnp.float32),
            pltpu.VMEM_SHARED((_NPAD, DH), jnp.float32),
            pltpu.VMEM_SHARED((_SPAD,), jnp.float32),
            pltpu.VMEM_SHARED((_SPAD,), jnp.float32),
            pltpu.SemaphoreType.DMA,
            pltpu.SemaphoreType.DMA,
            pltpu.SemaphoreType.DMA,
            pltpu.SemaphoreType.DMA,
        ],
    )
    return f(xcat, al, ar, src2d, dst2d, zeros2d, zeros1d)


def _mm0_body(x_ref, V_ref, alr_ref, xcat_ref):
    xb = x_ref[...]
    alr_ref[...] = jnp.dot(xb, V_ref[...], preferred_element_type=jnp.float32)
    xcat_ref[0] = xb[:, :DH]
    xcat_ref[1] = xb[:, DH:]


def _tc_proj(x, Vp):
    return pl.pallas_call(
        _mm0_body,
        grid=(N // 1000,),
        in_specs=[
            pl.BlockSpec((1000, D_IN), lambda i: (i, 0)),
            pl.BlockSpec((D_IN, D_IN), lambda i: (0, 0)),
        ],
        out_specs=[
            pl.BlockSpec((1000, D_IN), lambda i: (i, 0)),
            pl.BlockSpec((2, 1000, DH), lambda i: (0, i, 0)),
        ],
        out_shape=[
            jax.ShapeDtypeStruct((N, D_IN), jnp.float32),
            jax.ShapeDtypeStruct((2, N, DH), jnp.float32),
        ],
    )(x, Vp)


def _mm1_body(agg_ref, invs_ref, W1_ref, b1_ref, W2l_ref, W2r_ref,
              p2_ref, r2_ref):
    agg = jnp.concatenate([agg_ref[0], agg_ref[1]], axis=-1) * invs_ref[...]
    h2 = jnp.maximum(jnp.dot(agg, W1_ref[...],
                             preferred_element_type=jnp.float32)
                     + b1_ref[...], 0.0)
    p2 = jnp.dot(h2, W2l_ref[...], preferred_element_type=jnp.float32)
    p2_ref[0] = p2[:, :DH]
    p2_ref[1] = p2[:, DH:]
    r2_ref[...] = jnp.dot(h2, W2r_ref[...], preferred_element_type=jnp.float32)


def _tc_layer1(agg1p, inv_s, W1, b1, W2l, W2r):
    return pl.pallas_call(
        _mm1_body,
        grid=(_NPAD // _R,),
        in_specs=[
            pl.BlockSpec((2, _R, DH), lambda i: (0, i, 0)),
            pl.BlockSpec((_R, 1), lambda i: (i, 0)),
            pl.BlockSpec((D_IN, D_H1), lambda i: (0, 0)),
            pl.BlockSpec((1, D_H1), lambda i: (0, 0)),
            pl.BlockSpec((D_H1, D_H2), lambda i: (0, 0)),
            pl.BlockSpec((D_H1, D_H2), lambda i: (0, 0)),
        ],
        out_specs=[
            pl.BlockSpec((2, _R, DH), lambda i: (0, i, 0)),
            pl.BlockSpec((_R, D_H2), lambda i: (i, 0)),
        ],
        out_shape=[
            jax.ShapeDtypeStruct((2, _NPAD, DH), jnp.float32),
            jax.ShapeDtypeStruct((_NPAD, D_H2), jnp.float32),
        ],
    )(agg1p, inv_s, W1, b1, W2l, W2r)


def _mm2_body(agg_ref, invd_ref, r_ref, b_ref, Wl_ref, Wr_ref,
              p_ref, rn_ref):
    agg = jnp.concatenate([agg_ref[0], agg_ref[1]], axis=-1)
    out = jnp.maximum(agg * invd_ref[...] + r_ref[...] + b_ref[...], 0.0)
    p = jnp.dot(out, Wl_ref[...], preferred_element_type=jnp.float32)
    p_ref[0] = p[:, :DH]
    p_ref[1] = p[:, DH:]
    rn_ref[...] = jnp.dot(out, Wr_ref[...], preferred_element_type=jnp.float32)


def _tc_layer2(agg2p, invdeg, r2, b2, W3l, W3r):
    return pl.pallas_call(
        _mm2_body,
        grid=(_NPAD // _R,),
        in_specs=[
            pl.BlockSpec((2, _R, DH), lambda i: (0, i, 0)),
            pl.BlockSpec((_R, 1), lambda i: (i, 0)),
            pl.BlockSpec((_R, D_H2), lambda i: (i, 0)),
            pl.BlockSpec((1, D_H2), lambda i: (0, 0)),
            pl.BlockSpec((D_H2, D_OUT), lambda i: (0, 0)),
            pl.BlockSpec((D_H2, D_OUT), lambda i: (0, 0)),
        ],
        out_specs=[
            pl.BlockSpec((2, _R, DH), lambda i: (0, i, 0)),
            pl.BlockSpec((_R, D_OUT), lambda i: (i, 0)),
        ],
        out_shape=[
            jax.ShapeDtypeStruct((2, _NPAD, DH), jnp.float32),
            jax.ShapeDtypeStruct((_NPAD, D_OUT), jnp.float32),
        ],
    )(agg2p, invdeg, r2, b2, W3l, W3r)


def _mm3_body(agg_ref, invd_ref, r_ref, b_ref, out_ref):
    agg = jnp.concatenate([agg_ref[0], agg_ref[1]], axis=-1)
    out_ref[...] = agg * invd_ref[...] + r_ref[...] + b_ref[...]


def _tc_layer3(agg3p, invdeg, r3, b3):
    return pl.pallas_call(
        _mm3_body,
        grid=(_NPAD // _R,),
        in_specs=[
            pl.BlockSpec((2, _R, DH), lambda i: (0, i, 0)),
            pl.BlockSpec((_R, 1), lambda i: (i, 0)),
            pl.BlockSpec((_R, D_OUT), lambda i: (i, 0)),
            pl.BlockSpec((1, D_OUT), lambda i: (0, 0)),
        ],
        out_specs=pl.BlockSpec((_R, D_OUT), lambda i: (i, 0)),
        out_shape=jax.ShapeDtypeStruct((_NPAD, D_OUT), jnp.float32),
    )(agg3p, invdeg, r3, b3)


def kernel(x, edge_index, W1, a1_src, a1_dst, b1, W2l, W2r, b2, W3l, W3r, b3):
    src = edge_index[0]
    dst = edge_index[1]
    pad = _EROWS * _CH - E
    src2d = jnp.pad(src, (0, pad)).reshape(_EROWS, _CH)
    dst2d = jnp.pad(dst, (0, pad), constant_values=N).reshape(_EROWS, _CH)
    zeros2d = jnp.zeros((_NPAD, DH), jnp.float32)
    zeros1d = jnp.zeros((_SPAD,), jnp.float32)

    va = W1 @ a1_src
    vb = W1 @ a1_dst
    Vp = jnp.zeros((D_IN, D_IN), jnp.float32)
    Vp = Vp.at[:, 0].set(va).at[:, 1].set(vb)
    alr, xcat = _tc_proj(x, Vp)
    al = alr[:, 0]
    ar = alr[:, 1]

    agg1p, s0, d0 = _gat_sc(xcat.reshape(2 * N, DH), al, ar,
                            src2d, dst2d, zeros2d, zeros1d)
    inv_s = (1.0 / (s0[:_NPAD] + 1e-16))[:, None]
    invdeg = (1.0 / jnp.maximum(d0[:_NPAD], 1.0))[:, None]

    p2, r2 = _tc_layer1(agg1p, inv_s, W1, b1[None, :], W2l, W2r)

    agg2p = _spmm_sc(p2.reshape(2 * _NPAD, DH), src2d, dst2d, zeros2d)
    p3, r3 = _tc_layer2(agg2p, invdeg, r2, b2[None, :], W3l, W3r)

    agg3p = _spmm_sc(p3.reshape(2 * _NPAD, DH), src2d, dst2d, zeros2d)
    out3 = _tc_layer3(agg3p, invdeg, r3, b3[None, :])
    return out3[:N]
